# Initial kernel scaffold; baseline (speedup 1.0000x reference)
import jax
import jax.numpy as jnp
from jax.experimental import pallas as pl


def kernel(x_reals, x_cats, W_real, b_real, tables, pe):
    raise NotImplementedError("write your pallas kernel here")


# speedup vs baseline: 37.2512x; 37.2512x over previous
import functools

import jax
import jax.numpy as jnp
from jax import lax
from jax.experimental import pallas as pl
from jax.experimental.pallas import tpu as pltpu
from jax.experimental.pallas import tpu_sc as plsc

NC, NS = 2, 16
NW = NC * NS
CT = 128


def _sc_cat_sum(tables_flat, idx_chunks, n_tokens):
    FV, D = tables_flat.shape
    _, n_chunks, F, _ = idx_chunks.shape
    tpw = n_tokens // NW
    mesh = plsc.VectorSubcoreMesh(
        core_axis_name="c", subcore_axis_name="s", num_cores=NC, num_subcores=NS
    )

    @functools.partial(
        pl.kernel,
        out_type=jax.ShapeDtypeStruct((n_tokens, D), jnp.float32),
        mesh=mesh,
        scratch_types=[
            pltpu.VMEM((F, CT), jnp.int32),
            pltpu.VMEM((CT, D), jnp.float32),
            pltpu.SemaphoreType.DMA,
        ],
    )
    def k(tab_hbm, idx_hbm, out_hbm, idx_v, acc, sem):
        wid = lax.axis_index("s") * NC + lax.axis_index("c")

        def chunk_body(c, carry):
            base = wid * tpw + c * CT
            pltpu.sync_copy(idx_hbm.at[wid, c], idx_v)
            pltpu.async_copy(tab_hbm.at[idx_v.at[0]], acc, sem).wait()

            def field_body(f, carry2):
                pltpu.async_copy(tab_hbm.at[idx_v.at[f]], acc, sem, add=True).wait()
                return carry2

            lax.fori_loop(1, F, field_body, 0)
            pltpu.sync_copy(acc, out_hbm.at[pl.ds(base, CT)])
            return carry

        lax.fori_loop(0, n_chunks, chunk_body, 0)

    return k(tables_flat, idx_chunks)


def _tc_dense(x_reals, W_real, b_real, pe_s, cat_sum):
    B, S, R = x_reals.shape
    D = W_real.shape[0]
    BB = 8

    def body(x_ref, w_ref, b_ref, pe_ref, cat_ref, o_ref):
        x = x_ref[...].reshape(BB * S, R)
        y = lax.dot_general(
            x, w_ref[...], (((1,), (1,)), ((), ())),
            preferred_element_type=jnp.float32,
        )
        o_ref[...] = y.reshape(BB, S, D) + b_ref[...] + pe_ref[...] + cat_ref[...]

    return pl.pallas_call(
        body,
        grid=(B // BB,),
        in_specs=[
            pl.BlockSpec((BB, S, R), lambda i: (i, 0, 0)),
            pl.BlockSpec((D, R), lambda i: (0, 0)),
            pl.BlockSpec((D,), lambda i: (0,)),
            pl.BlockSpec((S, D), lambda i: (0, 0)),
            pl.BlockSpec((BB, S, D), lambda i: (i, 0, 0)),
        ],
        out_specs=pl.BlockSpec((BB, S, D), lambda i: (i, 0, 0)),
        out_shape=jax.ShapeDtypeStruct((B, S, D), jnp.float32),
    )(x_reals, W_real, b_real, pe_s, cat_sum)


def kernel(x_reals, x_cats, W_real, b_real, tables, pe):
    B, S, R = x_reals.shape
    F, V, D = tables.shape
    n_tokens = B * S
    tpw = n_tokens // NW
    n_chunks = tpw // CT

    tables_flat = tables.reshape(F * V, D)
    idx = x_cats.reshape(n_tokens, F) + jnp.arange(F, dtype=jnp.int32) * V
    idx_chunks = idx.reshape(NW, n_chunks, CT, F).transpose(0, 1, 3, 2)

    cat_sum = _sc_cat_sum(tables_flat, idx_chunks, n_tokens)
    pe_s = pe[0, :S]
    return _tc_dense(x_reals, W_real, b_real, pe_s, cat_sum.reshape(B, S, D))


# speedup vs baseline: 58.6144x; 1.5735x over previous
import functools

import jax
import jax.numpy as jnp
from jax import lax
from jax.experimental import pallas as pl
from jax.experimental.pallas import tpu as pltpu
from jax.experimental.pallas import tpu_sc as plsc

NC, NS = 2, 16
NW = NC * NS
CT = 128


def _sc_cat_sum(tables_flat, idx_chunks, n_tokens):
    FV, D = tables_flat.shape
    _, n_chunks, F, _ = idx_chunks.shape
    tpw = n_tokens // NW
    mesh = plsc.VectorSubcoreMesh(
        core_axis_name="c", subcore_axis_name="s", num_cores=NC, num_subcores=NS
    )

    @functools.partial(
        pl.kernel,
        out_type=jax.ShapeDtypeStruct((n_tokens, D), jnp.float32),
        mesh=mesh,
        scratch_types=[
            pltpu.VMEM((F, CT), jnp.int32),
            pltpu.VMEM((CT, D), jnp.float32),
            pltpu.SemaphoreType.DMA,
        ],
    )
    def k(tab_hbm, idx_hbm, out_hbm, idx_v, acc, sem):
        wid = lax.axis_index("s") * NC + lax.axis_index("c")

        def chunk_body(c, carry):
            base = wid * tpw + c * CT
            pltpu.sync_copy(idx_hbm.at[wid, c], idx_v)
            pltpu.async_copy(tab_hbm.at[idx_v.at[0]], acc, sem).wait()

            def fire(f, carry2):
                pltpu.async_copy(tab_hbm.at[idx_v.at[f]], acc, sem, add=True)
                return carry2

            lax.fori_loop(1, F, fire, 0)

            def drain(f, carry2):
                pltpu.make_async_copy(tab_hbm.at[idx_v.at[0]], acc, sem).wait()
                return carry2

            lax.fori_loop(1, F, drain, 0)
            pltpu.sync_copy(acc, out_hbm.at[pl.ds(base, CT)])
            return carry

        lax.fori_loop(0, n_chunks, chunk_body, 0)

    return k(tables_flat, idx_chunks)


def _tc_dense(x_reals, W_real, b_real, pe_s, cat_sum):
    B, S, R = x_reals.shape
    D = W_real.shape[0]
    BB = 8

    def body(x_ref, w_ref, b_ref, pe_ref, cat_ref, o_ref):
        x = x_ref[...].reshape(BB * S, R)
        y = lax.dot_general(
            x, w_ref[...], (((1,), (1,)), ((), ())),
            preferred_element_type=jnp.float32,
        )
        o_ref[...] = y.reshape(BB, S, D) + b_ref[...] + pe_ref[...] + cat_ref[...]

    return pl.pallas_call(
        body,
        grid=(B // BB,),
        in_specs=[
            pl.BlockSpec((BB, S, R), lambda i: (i, 0, 0)),
            pl.BlockSpec((D, R), lambda i: (0, 0)),
            pl.BlockSpec((D,), lambda i: (0,)),
            pl.BlockSpec((S, D), lambda i: (0, 0)),
            pl.BlockSpec((BB, S, D), lambda i: (i, 0, 0)),
        ],
        out_specs=pl.BlockSpec((BB, S, D), lambda i: (i, 0, 0)),
        out_shape=jax.ShapeDtypeStruct((B, S, D), jnp.float32),
    )(x_reals, W_real, b_real, pe_s, cat_sum)


def kernel(x_reals, x_cats, W_real, b_real, tables, pe):
    B, S, R = x_reals.shape
    F, V, D = tables.shape
    n_tokens = B * S
    tpw = n_tokens // NW
    n_chunks = tpw // CT

    tables_flat = tables.reshape(F * V, D)
    idx = x_cats.reshape(n_tokens, F) + jnp.arange(F, dtype=jnp.int32) * V
    idx_chunks = idx.reshape(NW, n_chunks, CT, F).transpose(0, 1, 3, 2)

    cat_sum = _sc_cat_sum(tables_flat, idx_chunks, n_tokens)
    pe_s = pe[0, :S]
    return _tc_dense(x_reals, W_real, b_real, pe_s, cat_sum.reshape(B, S, D))


# speedup vs baseline: 61.5747x; 1.0505x over previous
import functools

import jax
import jax.numpy as jnp
from jax import lax
from jax.experimental import pallas as pl
from jax.experimental.pallas import tpu as pltpu
from jax.experimental.pallas import tpu_sc as plsc

NC, NS = 2, 16
NW = NC * NS
CT = 128


def _sc_cat_sum(tables_flat, idx_chunks, n_tokens):
    FV, D = tables_flat.shape
    _, n_chunks, F, _ = idx_chunks.shape
    tpw = n_tokens // NW
    mesh = plsc.VectorSubcoreMesh(
        core_axis_name="c", subcore_axis_name="s", num_cores=NC, num_subcores=NS
    )

    @functools.partial(
        pl.kernel,
        out_type=jax.ShapeDtypeStruct((n_tokens, D), jnp.float32),
        mesh=mesh,
        scratch_types=[
            pltpu.VMEM((F, CT), jnp.int32),
            pltpu.VMEM((F, CT), jnp.int32),
            pltpu.VMEM((CT, D), jnp.float32),
            pltpu.VMEM((CT, D), jnp.float32),
            pltpu.SemaphoreType.DMA,
            pltpu.SemaphoreType.DMA,
            pltpu.SemaphoreType.DMA,
            pltpu.SemaphoreType.DMA,
        ],
    )
    def k(tab_hbm, idx_hbm, out_hbm, idxA, idxB, accA, accB,
          sem_f0, sem_g, sem_idx, sem_out):
        wid = lax.axis_index("s") * NC + lax.axis_index("c")

        pltpu.sync_copy(idx_hbm.at[wid, 0], idxA)
        pltpu.async_copy(tab_hbm.at[idxA.at[0]], accA, sem_f0)

        def half(c, idxP, accP, idxQ, accQ):
            base = wid * tpw + c * CT
            pltpu.make_async_copy(tab_hbm.at[idxP.at[0]], accP, sem_f0).wait()

            def fire(f, carry):
                pltpu.async_copy(tab_hbm.at[idxP.at[f]], accP, sem_g, add=True)
                return carry

            lax.fori_loop(1, F, fire, 0)

            @pl.when(c + 1 < n_chunks)
            def _():
                pltpu.async_copy(idx_hbm.at[wid, c + 1], idxQ, sem_idx)

            @pl.when(c >= 1)
            def _():
                pltpu.make_async_copy(
                    accQ, out_hbm.at[pl.ds(0, CT)], sem_out).wait()

            def drain(f, carry):
                pltpu.make_async_copy(
                    tab_hbm.at[idxP.at[0]], accP, sem_g).wait()
                return carry

            lax.fori_loop(1, F, drain, 0)
            pltpu.async_copy(accP, out_hbm.at[pl.ds(base, CT)], sem_out)

            @pl.when(c + 1 < n_chunks)
            def _():
                pltpu.make_async_copy(
                    idx_hbm.at[wid, 0], idxQ, sem_idx).wait()
                pltpu.async_copy(tab_hbm.at[idxQ.at[0]], accQ, sem_f0)

        def pair(i, carry):
            half(2 * i, idxA, accA, idxB, accB)
            half(2 * i + 1, idxB, accB, idxA, accA)
            return carry

        lax.fori_loop(0, n_chunks // 2, pair, 0)
        pltpu.make_async_copy(accB, out_hbm.at[pl.ds(0, CT)], sem_out).wait()

    return k(tables_flat, idx_chunks)


def _tc_dense(x_reals, W_real, b_real, pe_s, cat_sum):
    B, S, R = x_reals.shape
    D = W_real.shape[0]
    BB = 8

    def body(x_ref, w_ref, b_ref, pe_ref, cat_ref, o_ref):
        x = x_ref[...].reshape(BB * S, R)
        y = lax.dot_general(
            x, w_ref[...], (((1,), (1,)), ((), ())),
            preferred_element_type=jnp.float32,
        )
        cat = cat_ref[...].astype(jnp.float32)
        o_ref[...] = y.reshape(BB, S, D) + b_ref[...] + pe_ref[...] + cat

    return pl.pallas_call(
        body,
        grid=(B // BB,),
        in_specs=[
            pl.BlockSpec((BB, S, R), lambda i: (i, 0, 0)),
            pl.BlockSpec((D, R), lambda i: (0, 0)),
            pl.BlockSpec((D,), lambda i: (0,)),
            pl.BlockSpec((S, D), lambda i: (0, 0)),
            pl.BlockSpec((BB, S, D), lambda i: (i, 0, 0)),
        ],
        out_specs=pl.BlockSpec((BB, S, D), lambda i: (i, 0, 0)),
        out_shape=jax.ShapeDtypeStruct((B, S, D), jnp.float32),
    )(x_reals, W_real, b_real, pe_s, cat_sum)


def kernel(x_reals, x_cats, W_real, b_real, tables, pe):
    B, S, R = x_reals.shape
    F, V, D = tables.shape
    n_tokens = B * S
    tpw = n_tokens // NW
    n_chunks = tpw // CT

    tables_flat = tables.reshape(F * V, D)
    idx = x_cats.reshape(n_tokens, F) + jnp.arange(F, dtype=jnp.int32) * V
    idx_chunks = idx.reshape(NW, n_chunks, CT, F).transpose(0, 1, 3, 2)

    cat_sum = _sc_cat_sum(tables_flat, idx_chunks, n_tokens)
    pe_s = pe[0, :S]
    return _tc_dense(x_reals, W_real, b_real, pe_s, cat_sum.reshape(B, S, D))


# speedup vs baseline: 64.8477x; 1.0532x over previous
import functools

import jax
import jax.numpy as jnp
from jax import lax
from jax.experimental import pallas as pl
from jax.experimental.pallas import tpu as pltpu
from jax.experimental.pallas import tpu_sc as plsc

NC, NS = 2, 16
CT = 64


def _sc_cat_sum(tables_flat, idx_chunks, n_tokens, V):
    FV, D = tables_flat.shape
    _, _, n_chunks, F2, _ = idx_chunks.shape
    tpw = n_tokens // NS
    mesh = plsc.VectorSubcoreMesh(
        core_axis_name="c", subcore_axis_name="s", num_cores=NC, num_subcores=NS
    )

    @functools.partial(
        pl.kernel,
        out_type=jax.ShapeDtypeStruct((NC, n_tokens, D), jnp.float32),
        mesh=mesh,
        scratch_types=[
            pltpu.VMEM_SHARED((F2 * V, D), jnp.float32),
            pltpu.VMEM((F2, CT), jnp.int32),
            pltpu.VMEM((F2, CT), jnp.int32),
            pltpu.VMEM((CT, D), jnp.float32),
            pltpu.VMEM((CT, D), jnp.float32),
            pltpu.SemaphoreType.DMA,
            pltpu.SemaphoreType.DMA,
            pltpu.SemaphoreType.DMA,
            pltpu.SemaphoreType.DMA,
        ],
    )
    def k(tab_hbm, idx_hbm, out_hbm, shared, idxA, idxB, accA, accB,
          sem_f0, sem_g, sem_idx, sem_out):
        cid = lax.axis_index("c")
        sid = lax.axis_index("s")

        @pl.when(sid < F2)
        def _():
            pltpu.sync_copy(
                tab_hbm.at[pl.ds(cid * F2 * V + sid * V, V)],
                shared.at[pl.ds(sid * V, V)],
            )

        plsc.subcore_barrier()

        pltpu.sync_copy(idx_hbm.at[cid, sid, 0], idxA)
        pltpu.async_copy(shared.at[idxA.at[0]], accA, sem_f0)

        def half(c, idxP, accP, idxQ, accQ):
            base = sid * tpw + c * CT
            pltpu.make_async_copy(shared.at[idxP.at[0]], accP, sem_f0).wait()

            def fire(f, carry):
                pltpu.async_copy(shared.at[idxP.at[f]], accP, sem_g, add=True)
                return carry

            lax.fori_loop(1, F2, fire, 0)

            @pl.when(c + 1 < n_chunks)
            def _():
                pltpu.async_copy(idx_hbm.at[cid, sid, c + 1], idxQ, sem_idx)

            @pl.when(c >= 1)
            def _():
                pltpu.make_async_copy(
                    accQ, out_hbm.at[cid, pl.ds(0, CT)], sem_out).wait()

            def drain(f, carry):
                pltpu.make_async_copy(
                    shared.at[idxP.at[0]], accP, sem_g).wait()
                return carry

            lax.fori_loop(1, F2, drain, 0)
            pltpu.async_copy(accP, out_hbm.at[cid, pl.ds(base, CT)], sem_out)

            @pl.when(c + 1 < n_chunks)
            def _():
                pltpu.make_async_copy(
                    idx_hbm.at[cid, sid, 0], idxQ, sem_idx).wait()
                pltpu.async_copy(shared.at[idxQ.at[0]], accQ, sem_f0)

        def pair(i, carry):
            half(2 * i, idxA, accA, idxB, accB)
            half(2 * i + 1, idxB, accB, idxA, accA)
            return carry

        lax.fori_loop(0, n_chunks // 2, pair, 0)
        pltpu.make_async_copy(
            accB, out_hbm.at[0, pl.ds(0, CT)], sem_out).wait()

    return k(tables_flat, idx_chunks)


def _tc_dense(x_reals, W_real, b_real, pe_s, cat_sum):
    B, S, R = x_reals.shape
    D = W_real.shape[0]
    BB = 8

    def body(x_ref, w_ref, b_ref, pe_ref, cat_ref, o_ref):
        x = x_ref[...].reshape(BB * S, R)
        y = lax.dot_general(
            x, w_ref[...], (((1,), (1,)), ((), ())),
            preferred_element_type=jnp.float32,
        )
        cat = cat_ref[0] + cat_ref[1]
        o_ref[...] = y.reshape(BB, S, D) + b_ref[...] + pe_ref[...] + cat

    return pl.pallas_call(
        body,
        grid=(B // BB,),
        in_specs=[
            pl.BlockSpec((BB, S, R), lambda i: (i, 0, 0)),
            pl.BlockSpec((D, R), lambda i: (0, 0)),
            pl.BlockSpec((D,), lambda i: (0,)),
            pl.BlockSpec((S, D), lambda i: (0, 0)),
            pl.BlockSpec((NC, BB, S, D), lambda i: (0, i, 0, 0)),
        ],
        out_specs=pl.BlockSpec((BB, S, D), lambda i: (i, 0, 0)),
        out_shape=jax.ShapeDtypeStruct((B, S, D), jnp.float32),
    )(x_reals, W_real, b_real, pe_s, cat_sum)


def kernel(x_reals, x_cats, W_real, b_real, tables, pe):
    B, S, R = x_reals.shape
    F, V, D = tables.shape
    F2 = F // 2
    n_tokens = B * S
    tpw = n_tokens // NS
    n_chunks = tpw // CT

    tables_flat = tables.reshape(F * V, D)
    idx = x_cats.reshape(n_tokens, F) + (jnp.arange(F, dtype=jnp.int32) % F2) * V
    idx_chunks = (
        idx.T.reshape(NC, F2, NS, n_chunks, CT).transpose(0, 2, 3, 1, 4)
    )

    cat_sum = _sc_cat_sum(tables_flat, idx_chunks, n_tokens, V)
    pe_s = pe[0, :S]
    return _tc_dense(x_reals, W_real, b_real, pe_s,
                     cat_sum.reshape(NC, B, S, D))
